# Initial kernel scaffold; baseline (speedup 1.0000x reference)
import jax
import jax.numpy as jnp
from jax.experimental import pallas as pl


def kernel(embs, seq_lengths, pos_table):
    raise NotImplementedError("write your pallas kernel here")


# speedup vs baseline: 2.5823x; 2.5823x over previous
import functools

import jax
import jax.numpy as jnp
from jax import lax
from jax.experimental import pallas as pl
from jax.experimental.pallas import tpu as pltpu
from jax.experimental.pallas import tpu_sc as plsc

NC = 2
NS = 16
LANES = 16
NW = NC * NS


def _body(embs_hbm, seq_hbm, pos_hbm, out_hbm, pos_v, seq_v, buf):
    batch, seq_len, d_model = embs_hbm.shape
    items = batch // NW
    wid = lax.axis_index("s") * NC + lax.axis_index("c")
    base = wid * items

    pos_rows = pos_v.shape[0]
    pltpu.sync_copy(pos_hbm.at[pl.ds(0, pos_rows)], pos_v)
    pltpu.sync_copy(seq_hbm.at[pl.ds(base, items)], seq_v)

    vregs_per_row = d_model // LANES

    def group_body(g, carry):
        seq16 = seq_v[pl.ds(g * LANES, LANES)]
        for lane in range(LANES):
            i = g * LANES + lane
            n = jnp.minimum(seq16[lane], seq_len)
            pltpu.sync_copy(embs_hbm.at[base + i], buf)

            def row_body(j, c):
                for k in range(vregs_per_row):
                    sl = pl.ds(k * LANES, LANES)
                    buf[j, sl] += pos_v[j + 1, sl]
                return c

            lax.fori_loop(0, n, row_body, 0)
            pltpu.sync_copy(buf, out_hbm.at[base + i])
        return carry

    lax.fori_loop(0, items // LANES, group_body, 0)


@jax.jit
def kernel(embs, seq_lengths, pos_table):
    batch, seq_len, d_model = embs.shape
    mesh = plsc.VectorSubcoreMesh(
        core_axis_name="c", subcore_axis_name="s", num_cores=NC, num_subcores=NS
    )
    run = pl.kernel(
        _body,
        out_type=jax.ShapeDtypeStruct((batch, seq_len, d_model), embs.dtype),
        mesh=mesh,
        scratch_types=[
            pltpu.VMEM((seq_len + 8, d_model), jnp.float32),
            pltpu.VMEM((batch // NW,), jnp.int32),
            pltpu.VMEM((seq_len, d_model), jnp.float32),
        ],
    )
    return run(embs, seq_lengths.astype(jnp.int32), pos_table)
